# Initial kernel scaffold; baseline (speedup 1.0000x reference)
import jax
import jax.numpy as jnp
from jax.experimental import pallas as pl


def kernel(x, edge_index, batch, W1, b1, g1, be1, W2, b2, g2, be2, Wp, bp):
    raise NotImplementedError("write your pallas kernel here")


# speedup vs baseline: 10.3418x; 10.3418x over previous
import functools

import jax
import jax.numpy as jnp
from jax import lax
from jax.experimental import pallas as pl
from jax.experimental.pallas import tpu as pltpu
from jax.experimental.pallas import tpu_sc as plsc

_NC = 2
_NS = 16
_NW = _NC * _NS
_CH = 128
_NG = 64
_EPS = 1e-5


def _mesh():
    return plsc.VectorSubcoreMesh(
        core_axis_name="c", subcore_axis_name="s",
        num_cores=_NC, num_subcores=_NS)


@functools.lru_cache(maxsize=None)
def _sc_degree(np_, cpt):
    @functools.partial(
        pl.kernel, mesh=_mesh(),
        out_type=jax.ShapeDtypeStruct((_NC, np_, _CH), jnp.float32),
        scratch_types=[
            pltpu.VMEM((_CH,), jnp.int32),
            pltpu.VMEM((_CH,), jnp.int32),
            pltpu.VMEM((_CH, _CH), jnp.float32),
            pltpu.VMEM_SHARED((np_, _CH), jnp.float32),
            pltpu.SemaphoreType.DMA,
            pltpu.SemaphoreType.DMA,
        ],
    )
    def deg_kernel(dst_hbm, zeros_hbm, ones_hbm, out_hbm, d_a, d_b, ones_v,
                   acc, ssa, ssb):
        c = lax.axis_index("c")
        s = lax.axis_index("s")
        wid = s * _NC + c
        rpt = np_ // _NS
        r0 = s * rpt
        pltpu.sync_copy(ones_hbm, ones_v)
        pltpu.sync_copy(zeros_hbm, acc.at[pl.ds(r0, rpt)])
        plsc.subcore_barrier()

        half = cpt // 2

        def body(jj, carry):
            c0 = jj * 2
            c1 = c0 + 1

            @pl.when(jj > 0)
            def _():
                pltpu.make_async_copy(ones_v, acc.at[d_a], ssa).wait()

            pltpu.sync_copy(dst_hbm.at[wid, c0], d_a)
            pltpu.async_copy(ones_v, acc.at[d_a], ssa, add=True)

            @pl.when(jj > 0)
            def _():
                pltpu.make_async_copy(ones_v, acc.at[d_b], ssb).wait()

            pltpu.sync_copy(dst_hbm.at[wid, c1], d_b)
            pltpu.async_copy(ones_v, acc.at[d_b], ssb, add=True)
            return carry

        lax.fori_loop(0, half, body, 0)
        pltpu.make_async_copy(ones_v, acc.at[d_a], ssa).wait()
        pltpu.make_async_copy(ones_v, acc.at[d_b], ssb).wait()
        plsc.subcore_barrier()
        pltpu.sync_copy(acc.at[pl.ds(r0, rpt)], out_hbm.at[c, pl.ds(r0, rpt)])

    return deg_kernel


@functools.lru_cache(maxsize=None)
def _sc_agg(np_, d, cpt):
    @functools.partial(
        pl.kernel, mesh=_mesh(),
        out_type=jax.ShapeDtypeStruct((_NC, np_, d), jnp.float32),
        scratch_types=[
            pltpu.VMEM((cpt, _CH), jnp.int32),
            pltpu.VMEM((_CH,), jnp.int32),
            pltpu.VMEM((_CH,), jnp.int32),
            pltpu.VMEM((_CH, d), jnp.float32),
            pltpu.VMEM((_CH, d), jnp.float32),
            pltpu.SemaphoreType.DMA,
            pltpu.SemaphoreType.DMA,
            pltpu.SemaphoreType.DMA,
            pltpu.SemaphoreType.DMA,
            pltpu.VMEM_SHARED((np_, d), jnp.float32),
        ],
    )
    def agg_kernel(xs_hbm, src_hbm, dst_hbm, zeros_hbm, out_hbm,
                   src_v, d_a, d_b, rows_a, rows_b, gsa, gsb, ssa, ssb, acc):
        c = lax.axis_index("c")
        s = lax.axis_index("s")
        wid = s * _NC + c
        rpt = np_ // _NS
        r0 = s * rpt
        pltpu.sync_copy(src_hbm.at[wid], src_v)
        pltpu.sync_copy(zeros_hbm, acc.at[pl.ds(r0, rpt)])
        plsc.subcore_barrier()

        half = cpt // 2
        pltpu.async_copy(xs_hbm.at[src_v.at[0]], rows_a, gsa)

        def body(jj, carry):
            c0 = jj * 2
            c1 = c0 + 1
            pltpu.make_async_copy(xs_hbm.at[src_v.at[c0]], rows_a, gsa).wait()
            pltpu.sync_copy(dst_hbm.at[wid, c0], d_a)
            pltpu.async_copy(rows_a, acc.at[d_a], ssa, add=True)

            @pl.when(jj > 0)
            def _():
                pltpu.make_async_copy(rows_b, acc.at[d_b], ssb).wait()

            pltpu.async_copy(xs_hbm.at[src_v.at[c1]], rows_b, gsb)
            pltpu.make_async_copy(rows_a, acc.at[d_a], ssa).wait()

            @pl.when(jj < half - 1)
            def _():
                pltpu.async_copy(xs_hbm.at[src_v.at[c0 + 2]], rows_a, gsa)

            pltpu.make_async_copy(xs_hbm.at[src_v.at[c1]], rows_b, gsb).wait()
            pltpu.sync_copy(dst_hbm.at[wid, c1], d_b)
            pltpu.async_copy(rows_b, acc.at[d_b], ssb, add=True)
            return carry

        lax.fori_loop(0, half, body, 0)
        pltpu.make_async_copy(rows_b, acc.at[d_b], ssb).wait()
        plsc.subcore_barrier()
        pltpu.sync_copy(acc.at[pl.ds(r0, rpt)], out_hbm.at[c, pl.ds(r0, rpt)])

    return agg_kernel


@functools.lru_cache(maxsize=None)
def _tc_prep(np_, n, din):
    def body(degp_ref, x_ref, dinv_ref, xs_ref):
        deg = degp_ref[0, :, 0:1] + degp_ref[1, :, 0:1] + 1.0
        rows = lax.broadcasted_iota(jnp.int32, (np_, 1), 0)
        dinv = jnp.where(rows < n, 1.0 / jnp.sqrt(deg), 0.0)
        dinv_ref[...] = dinv
        xs_ref[...] = dinv * x_ref[...]

    return pl.pallas_call(
        body,
        out_shape=(jax.ShapeDtypeStruct((np_, 1), jnp.float32),
                   jax.ShapeDtypeStruct((np_, din), jnp.float32)),
    )


@functools.lru_cache(maxsize=None)
def _tc_mid(np_, n, din, dh, de):
    def body(p_ref, xs_ref, dinv_ref, w1_ref, b1_ref, g1_ref, be1_ref,
             w2_ref, xs2_ref):
        dinv = dinv_ref[...]
        agg = dinv * (p_ref[0] + p_ref[1] + xs_ref[...])
        z = jnp.dot(agg.astype(jnp.bfloat16), w1_ref[...].astype(jnp.bfloat16),
                    preferred_element_type=jnp.float32)
        z = z + b1_ref[...]
        rows = lax.broadcasted_iota(jnp.int32, (np_, 1), 0)
        r = jnp.where(rows < n, jnp.maximum(z, 0.0), 0.0)
        mu = jnp.sum(r, axis=0, keepdims=True) * (1.0 / n)
        m2 = jnp.sum(r * r, axis=0, keepdims=True) * (1.0 / n)
        var = m2 - mu * mu
        h1 = (r - mu) * (g1_ref[...] * lax.rsqrt(var + _EPS)) + be1_ref[...]
        y2 = jnp.dot(h1.astype(jnp.bfloat16), w2_ref[...].astype(jnp.bfloat16),
                     preferred_element_type=jnp.float32)
        xs2_ref[...] = dinv * y2

    return pl.pallas_call(
        body,
        out_shape=jax.ShapeDtypeStruct((np_, de), jnp.float32),
    )


@functools.lru_cache(maxsize=None)
def _tc_final(np_, n, de, ng):
    def body(p_ref, xs2_ref, dinv_ref, b2_ref, g2_ref, be2_ref, batch_ref,
             wp_ref, bp_ref, out_ref, pooled):
        agg = dinv_ref[...] * (p_ref[0] + p_ref[1] + xs2_ref[...])
        z = agg + b2_ref[...]
        rows = lax.broadcasted_iota(jnp.int32, (np_, 1), 0)
        r = jnp.where(rows < n, jnp.maximum(z, 0.0), 0.0)
        mu = jnp.sum(r, axis=0, keepdims=True) * (1.0 / n)
        m2 = jnp.sum(r * r, axis=0, keepdims=True) * (1.0 / n)
        var = m2 - mu * mu
        h2 = (r - mu) * (g2_ref[...] * lax.rsqrt(var + _EPS)) + be2_ref[...]
        bids = batch_ref[...]

        def gbody(g, carry):
            m = jnp.where(bids == g, h2, -jnp.inf)
            pooled[pl.ds(g, 1), :] = jnp.max(m, axis=0, keepdims=True)
            return carry

        lax.fori_loop(0, ng, gbody, 0)
        out_ref[...] = (
            jnp.dot(pooled[...].astype(jnp.bfloat16),
                    wp_ref[...].astype(jnp.bfloat16),
                    preferred_element_type=jnp.float32)
            + bp_ref[...])

    return pl.pallas_call(
        body,
        out_shape=jax.ShapeDtypeStruct((ng, 1), jnp.float32),
        scratch_shapes=[pltpu.VMEM((ng, de), jnp.float32)],
    )


def kernel(x, edge_index, batch, W1, b1, g1, be1, W2, b2, g2, be2, Wp, bp):
    n, din = x.shape
    dh = W1.shape[1]
    de = W2.shape[1]
    e = edge_index.shape[1]

    rpt = -(-n // _NS)
    rpt = -(-rpt // 8) * 8
    np_ = rpt * _NS
    cpt = -(-e // (_NW * _CH))
    cpt = cpt + (cpt % 2)
    ep = _NW * cpt * _CH
    pad_rows = np_ - n

    x = x.astype(jnp.float32)
    src = edge_index[0]
    dst = edge_index[1]
    pad_e = ep - e
    src_p = jnp.concatenate([src, jnp.full((pad_e,), n, jnp.int32)])
    spread = n + jnp.arange(pad_e, dtype=jnp.int32) % jnp.int32(pad_rows)
    dst_p = jnp.concatenate([dst, spread])
    src3 = src_p.reshape(_NW, cpt, _CH)
    dst3 = dst_p.reshape(_NW, cpt, _CH)

    xp = jnp.zeros((np_, din), jnp.float32).at[:n].set(x)
    batch_p = jnp.concatenate(
        [batch.astype(jnp.int32), jnp.full((pad_rows,), _NG, jnp.int32)]
    ).reshape(np_, 1)

    zeros_d = jnp.zeros((rpt, din), jnp.float32)
    zeros_c = jnp.zeros((rpt, _CH), jnp.float32)
    ones_c = jnp.ones((_CH, _CH), jnp.float32)

    degp = _sc_degree(np_, cpt)(dst3, zeros_c, ones_c)
    dinv, xs = _tc_prep(np_, n, din)(degp, xp)
    p1 = _sc_agg(np_, din, cpt)(xs, src3, dst3, zeros_d)
    xs2 = _tc_mid(np_, n, din, dh, de)(
        p1, xs, dinv, W1, b1.reshape(1, dh), g1.reshape(1, dh),
        be1.reshape(1, dh), W2)
    p2 = _sc_agg(np_, de, cpt)(xs2, src3, dst3, zeros_d)
    out = _tc_final(np_, n, de, _NG)(
        p2, xs2, dinv, b2.reshape(1, de), g2.reshape(1, de),
        be2.reshape(1, de), batch_p, Wp, bp.reshape(1, 1))
    return out


# speedup vs baseline: 21.7425x; 2.1024x over previous
import functools

import jax
import jax.numpy as jnp
from jax import lax
from jax.experimental import pallas as pl
from jax.experimental.pallas import tpu as pltpu
from jax.experimental.pallas import tpu_sc as plsc

_NC = 2
_NS = 16
_NW = _NC * _NS
_CH = 128
_NG = 64
_EPS = 1e-5


def _mesh():
    return plsc.VectorSubcoreMesh(
        core_axis_name="c", subcore_axis_name="s",
        num_cores=_NC, num_subcores=_NS)


@functools.lru_cache(maxsize=None)
def _sc_degree(np_, cpt):
    @functools.partial(
        pl.kernel, mesh=_mesh(),
        out_type=jax.ShapeDtypeStruct((_NC, np_, _CH), jnp.float32),
        scratch_types=[
            pltpu.VMEM((_CH,), jnp.int32),
            pltpu.VMEM((_CH,), jnp.int32),
            pltpu.VMEM((_CH, _CH), jnp.float32),
            pltpu.VMEM_SHARED((np_, _CH), jnp.float32),
            pltpu.SemaphoreType.DMA,
            pltpu.SemaphoreType.DMA,
        ],
    )
    def deg_kernel(dst_hbm, zeros_hbm, ones_hbm, out_hbm, d_a, d_b, ones_v,
                   acc, ssa, ssb):
        c = lax.axis_index("c")
        s = lax.axis_index("s")
        wid = s * _NC + c
        rpt = np_ // _NS
        r0 = s * rpt
        pltpu.sync_copy(ones_hbm, ones_v)
        pltpu.sync_copy(zeros_hbm, acc.at[pl.ds(r0, rpt)])
        plsc.subcore_barrier()

        half = cpt // 2

        def body(jj, carry):
            c0 = jj * 2
            c1 = c0 + 1

            @pl.when(jj > 0)
            def _():
                pltpu.make_async_copy(ones_v, acc.at[d_a], ssa).wait()

            pltpu.sync_copy(dst_hbm.at[wid, c0], d_a)
            pltpu.async_copy(ones_v, acc.at[d_a], ssa, add=True)

            @pl.when(jj > 0)
            def _():
                pltpu.make_async_copy(ones_v, acc.at[d_b], ssb).wait()

            pltpu.sync_copy(dst_hbm.at[wid, c1], d_b)
            pltpu.async_copy(ones_v, acc.at[d_b], ssb, add=True)
            return carry

        lax.fori_loop(0, half, body, 0)
        pltpu.make_async_copy(ones_v, acc.at[d_a], ssa).wait()
        pltpu.make_async_copy(ones_v, acc.at[d_b], ssb).wait()
        plsc.subcore_barrier()
        pltpu.sync_copy(acc.at[pl.ds(r0, rpt)], out_hbm.at[c, pl.ds(r0, rpt)])

    return deg_kernel


@functools.lru_cache(maxsize=None)
def _sc_agg(np_, d, cpt):
    @functools.partial(
        pl.kernel, mesh=_mesh(),
        out_type=jax.ShapeDtypeStruct((_NC, np_, d), jnp.float32),
        scratch_types=[
            pltpu.VMEM((cpt, _CH), jnp.int32),
            pltpu.VMEM((_CH,), jnp.int32),
            pltpu.VMEM((_CH,), jnp.int32),
            pltpu.VMEM((_CH, d), jnp.float32),
            pltpu.VMEM((_CH, d), jnp.float32),
            pltpu.SemaphoreType.DMA,
            pltpu.SemaphoreType.DMA,
            pltpu.SemaphoreType.DMA,
            pltpu.SemaphoreType.DMA,
            pltpu.VMEM_SHARED((np_, d), jnp.float32),
        ],
    )
    def agg_kernel(xs_hbm, src_hbm, dst_hbm, zeros_hbm, out_hbm,
                   src_v, d_a, d_b, rows_a, rows_b, gsa, gsb, ssa, ssb, acc):
        c = lax.axis_index("c")
        s = lax.axis_index("s")
        wid = s * _NC + c
        rpt = np_ // _NS
        r0 = s * rpt
        pltpu.sync_copy(src_hbm.at[wid], src_v)
        pltpu.sync_copy(zeros_hbm, acc.at[pl.ds(r0, rpt)])
        plsc.subcore_barrier()

        half = cpt // 2
        pltpu.async_copy(xs_hbm.at[src_v.at[0]], rows_a, gsa)

        def body(jj, carry):
            c0 = jj * 2
            c1 = c0 + 1
            pltpu.make_async_copy(xs_hbm.at[src_v.at[c0]], rows_a, gsa).wait()
            pltpu.sync_copy(dst_hbm.at[wid, c0], d_a)
            pltpu.async_copy(rows_a, acc.at[d_a], ssa, add=True)

            @pl.when(jj > 0)
            def _():
                pltpu.make_async_copy(rows_b, acc.at[d_b], ssb).wait()

            pltpu.async_copy(xs_hbm.at[src_v.at[c1]], rows_b, gsb)
            pltpu.make_async_copy(rows_a, acc.at[d_a], ssa).wait()

            @pl.when(jj < half - 1)
            def _():
                pltpu.async_copy(xs_hbm.at[src_v.at[c0 + 2]], rows_a, gsa)

            pltpu.make_async_copy(xs_hbm.at[src_v.at[c1]], rows_b, gsb).wait()
            pltpu.sync_copy(dst_hbm.at[wid, c1], d_b)
            pltpu.async_copy(rows_b, acc.at[d_b], ssb, add=True)
            return carry

        lax.fori_loop(0, half, body, 0)
        pltpu.make_async_copy(rows_b, acc.at[d_b], ssb).wait()
        plsc.subcore_barrier()
        pltpu.sync_copy(acc.at[pl.ds(r0, rpt)], out_hbm.at[c, pl.ds(r0, rpt)])

    return agg_kernel


@functools.lru_cache(maxsize=None)
def _tc_prep(np_, n, din):
    def body(degp_ref, x_ref, dinv_ref, xs_ref):
        deg = degp_ref[0, :, 0:1] + degp_ref[1, :, 0:1] + 1.0
        rows = lax.broadcasted_iota(jnp.int32, (np_, 1), 0)
        dinv = jnp.where(rows < n, 1.0 / jnp.sqrt(deg), 0.0)
        dinv_ref[...] = dinv
        xs_ref[...] = dinv * x_ref[...]

    return pl.pallas_call(
        body,
        out_shape=(jax.ShapeDtypeStruct((np_, 1), jnp.float32),
                   jax.ShapeDtypeStruct((np_, din), jnp.float32)),
    )


@functools.lru_cache(maxsize=None)
def _tc_mid(np_, n, din, dh, de):
    def body(p_ref, xs_ref, dinv_ref, w1_ref, b1_ref, g1_ref, be1_ref,
             w2_ref, xs2_ref):
        dinv = dinv_ref[...]
        agg = dinv * (p_ref[0] + p_ref[1] + xs_ref[...])
        z = jnp.dot(agg.astype(jnp.bfloat16), w1_ref[...].astype(jnp.bfloat16),
                    preferred_element_type=jnp.float32)
        z = z + b1_ref[...]
        rows = lax.broadcasted_iota(jnp.int32, (np_, 1), 0)
        r = jnp.where(rows < n, jnp.maximum(z, 0.0), 0.0)
        mu = jnp.sum(r, axis=0, keepdims=True) * (1.0 / n)
        m2 = jnp.sum(r * r, axis=0, keepdims=True) * (1.0 / n)
        var = m2 - mu * mu
        h1 = (r - mu) * (g1_ref[...] * lax.rsqrt(var + _EPS)) + be1_ref[...]
        y2 = jnp.dot(h1.astype(jnp.bfloat16), w2_ref[...].astype(jnp.bfloat16),
                     preferred_element_type=jnp.float32)
        xs2_ref[...] = dinv * y2

    return pl.pallas_call(
        body,
        out_shape=jax.ShapeDtypeStruct((np_, de), jnp.float32),
    )


@functools.lru_cache(maxsize=None)
def _tc_final(np_, n, de, ng):
    def body(p_ref, xs2_ref, dinv_ref, b2_ref, g2_ref, be2_ref, batch_ref,
             wp_ref, bp_ref, out_ref, pooled):
        agg = dinv_ref[...] * (p_ref[0] + p_ref[1] + xs2_ref[...])
        z = agg + b2_ref[...]
        rows = lax.broadcasted_iota(jnp.int32, (np_, 1), 0)
        r = jnp.where(rows < n, jnp.maximum(z, 0.0), 0.0)
        mu = jnp.sum(r, axis=0, keepdims=True) * (1.0 / n)
        m2 = jnp.sum(r * r, axis=0, keepdims=True) * (1.0 / n)
        var = m2 - mu * mu
        h2 = (r - mu) * (g2_ref[...] * lax.rsqrt(var + _EPS)) + be2_ref[...]
        bids = batch_ref[...]

        def gbody(g, carry):
            m = jnp.where(bids == g, h2, -jnp.inf)
            pooled[pl.ds(g, 1), :] = jnp.max(m, axis=0, keepdims=True)
            return carry

        lax.fori_loop(0, ng, gbody, 0)
        out_ref[...] = (
            jnp.dot(pooled[...].astype(jnp.bfloat16),
                    wp_ref[...].astype(jnp.bfloat16),
                    preferred_element_type=jnp.float32)
            + bp_ref[...])

    return pl.pallas_call(
        body,
        out_shape=jax.ShapeDtypeStruct((ng, 1), jnp.float32),
        scratch_shapes=[pltpu.VMEM((ng, de), jnp.float32)],
    )


def kernel(x, edge_index, batch, W1, b1, g1, be1, W2, b2, g2, be2, Wp, bp):
    n, din = x.shape
    dh = W1.shape[1]
    de = W2.shape[1]
    e = edge_index.shape[1]

    rpt = -(-n // _NS)
    rpt = -(-rpt // 8) * 8
    np_ = rpt * _NS
    cpt = -(-e // (_NW * _CH))
    cpt = cpt + (cpt % 2)
    ep = _NW * cpt * _CH
    pad_rows = np_ - n

    x = x.astype(jnp.float32)
    src = edge_index[0]
    dst = edge_index[1]
    pad_e = ep - e
    spread = n + jnp.arange(pad_e, dtype=jnp.int32) % jnp.int32(pad_rows)
    src_p = jnp.concatenate([src, spread])
    dst_p = jnp.concatenate([dst, spread])
    src3 = src_p.reshape(_NW, cpt, _CH)
    dst3 = dst_p.reshape(_NW, cpt, _CH)

    xp = jnp.zeros((np_, din), jnp.float32).at[:n].set(x)
    batch_p = jnp.concatenate(
        [batch.astype(jnp.int32), jnp.full((pad_rows,), _NG, jnp.int32)]
    ).reshape(np_, 1)

    zeros_d = jnp.zeros((rpt, din), jnp.float32)
    zeros_c = jnp.zeros((rpt, _CH), jnp.float32)
    ones_c = jnp.ones((_CH, _CH), jnp.float32)

    degp = _sc_degree(np_, cpt)(dst3, zeros_c, ones_c)
    dinv, xs = _tc_prep(np_, n, din)(degp, xp)
    p1 = _sc_agg(np_, din, cpt)(xs, src3, dst3, zeros_d)
    xs2 = _tc_mid(np_, n, din, dh, de)(
        p1, xs, dinv, W1, b1.reshape(1, dh), g1.reshape(1, dh),
        be1.reshape(1, dh), W2)
    p2 = _sc_agg(np_, de, cpt)(xs2, src3, dst3, zeros_d)
    out = _tc_final(np_, n, de, _NG)(
        p2, xs2, dinv, b2.reshape(1, de), g2.reshape(1, de),
        be2.reshape(1, de), batch_p, Wp, bp.reshape(1, 1))
    return out


# speedup vs baseline: 25.2642x; 1.1620x over previous
import functools

import jax
import jax.numpy as jnp
from jax import lax
from jax.experimental import pallas as pl
from jax.experimental.pallas import tpu as pltpu
from jax.experimental.pallas import tpu_sc as plsc

_NC = 2
_NS = 16
_NW = _NC * _NS
_CH = 128
_NG = 64
_EPS = 1e-5


def _mesh():
    return plsc.VectorSubcoreMesh(
        core_axis_name="c", subcore_axis_name="s",
        num_cores=_NC, num_subcores=_NS)


@functools.lru_cache(maxsize=None)
def _sc_degree(np_, cpt):
    @functools.partial(
        pl.kernel, mesh=_mesh(),
        out_type=jax.ShapeDtypeStruct((_NC, np_, _CH), jnp.float32),
        scratch_types=[
            pltpu.VMEM((_CH,), jnp.int32),
            pltpu.VMEM((_CH,), jnp.int32),
            pltpu.VMEM((_CH, _CH), jnp.float32),
            pltpu.VMEM_SHARED((np_, _CH), jnp.float32),
            pltpu.SemaphoreType.DMA,
            pltpu.SemaphoreType.DMA,
        ],
    )
    def deg_kernel(dst_hbm, zeros_hbm, ones_hbm, out_hbm, d_a, d_b, ones_v,
                   acc, ssa, ssb):
        c = lax.axis_index("c")
        s = lax.axis_index("s")
        wid = s * _NC + c
        rpt = np_ // _NS
        r0 = s * rpt
        pltpu.sync_copy(ones_hbm, ones_v)
        pltpu.sync_copy(zeros_hbm, acc.at[pl.ds(r0, rpt)])
        plsc.subcore_barrier()

        half = cpt // 2

        def body(jj, carry):
            c0 = jj * 2
            c1 = c0 + 1

            @pl.when(jj > 0)
            def _():
                pltpu.make_async_copy(ones_v, acc.at[d_a], ssa).wait()

            pltpu.sync_copy(dst_hbm.at[wid, c0], d_a)
            pltpu.async_copy(ones_v, acc.at[d_a], ssa, add=True)

            @pl.when(jj > 0)
            def _():
                pltpu.make_async_copy(ones_v, acc.at[d_b], ssb).wait()

            pltpu.sync_copy(dst_hbm.at[wid, c1], d_b)
            pltpu.async_copy(ones_v, acc.at[d_b], ssb, add=True)
            return carry

        lax.fori_loop(0, half, body, 0)
        pltpu.make_async_copy(ones_v, acc.at[d_a], ssa).wait()
        pltpu.make_async_copy(ones_v, acc.at[d_b], ssb).wait()
        plsc.subcore_barrier()
        pltpu.sync_copy(acc.at[pl.ds(r0, rpt)], out_hbm.at[c, pl.ds(r0, rpt)])

    return deg_kernel


@functools.lru_cache(maxsize=None)
def _sc_agg(np_, d, cpt):
    @functools.partial(
        pl.kernel, mesh=_mesh(),
        out_type=jax.ShapeDtypeStruct((_NC, np_, d), jnp.float32),
        scratch_types=[
            pltpu.VMEM((cpt, _CH), jnp.int32),
            pltpu.VMEM((_CH,), jnp.int32),
            pltpu.VMEM((_CH,), jnp.int32),
            pltpu.VMEM((_CH, d), jnp.float32),
            pltpu.VMEM((_CH, d), jnp.float32),
            pltpu.SemaphoreType.DMA,
            pltpu.SemaphoreType.DMA,
            pltpu.SemaphoreType.DMA,
            pltpu.SemaphoreType.DMA,
            pltpu.VMEM_SHARED((np_, d), jnp.float32),
        ],
    )
    def agg_kernel(xs_hbm, src_hbm, dst_hbm, zeros_hbm, out_hbm,
                   src_v, d_a, d_b, rows_a, rows_b, gsa, gsb, ssa, ssb, acc):
        c = lax.axis_index("c")
        s = lax.axis_index("s")
        wid = s * _NC + c
        rpt = np_ // _NS
        r0 = s * rpt
        pltpu.sync_copy(src_hbm.at[wid], src_v)
        pltpu.sync_copy(zeros_hbm, acc.at[pl.ds(r0, rpt)])
        plsc.subcore_barrier()

        half = cpt // 2
        pltpu.async_copy(xs_hbm.at[src_v.at[0]], rows_a, gsa)

        def body(jj, carry):
            c0 = jj * 2
            c1 = c0 + 1
            pltpu.make_async_copy(xs_hbm.at[src_v.at[c0]], rows_a, gsa).wait()
            pltpu.sync_copy(dst_hbm.at[wid, c0], d_a)
            pltpu.async_copy(rows_a, acc.at[d_a], ssa, add=True)

            @pl.when(jj > 0)
            def _():
                pltpu.make_async_copy(rows_b, acc.at[d_b], ssb).wait()

            pltpu.async_copy(xs_hbm.at[src_v.at[c1]], rows_b, gsb)
            pltpu.make_async_copy(rows_a, acc.at[d_a], ssa).wait()

            @pl.when(jj < half - 1)
            def _():
                pltpu.async_copy(xs_hbm.at[src_v.at[c0 + 2]], rows_a, gsa)

            pltpu.make_async_copy(xs_hbm.at[src_v.at[c1]], rows_b, gsb).wait()
            pltpu.sync_copy(dst_hbm.at[wid, c1], d_b)
            pltpu.async_copy(rows_b, acc.at[d_b], ssb, add=True)
            return carry

        lax.fori_loop(0, half, body, 0)
        pltpu.make_async_copy(rows_b, acc.at[d_b], ssb).wait()
        plsc.subcore_barrier()
        pltpu.sync_copy(acc.at[pl.ds(r0, rpt)], out_hbm.at[c, pl.ds(r0, rpt)])

    return agg_kernel


@functools.lru_cache(maxsize=None)
def _tc_prep(np_, n, din):
    def body(degp_ref, x_ref, dinv_ref, xs_ref):
        deg = degp_ref[0, :, 0:1] + degp_ref[1, :, 0:1] + 1.0
        rows = lax.broadcasted_iota(jnp.int32, (np_, 1), 0)
        dinv = jnp.where(rows < n, 1.0 / jnp.sqrt(deg), 0.0)
        dinv_ref[...] = dinv
        xs_ref[...] = dinv * x_ref[...]

    return pl.pallas_call(
        body,
        out_shape=(jax.ShapeDtypeStruct((np_, 1), jnp.float32),
                   jax.ShapeDtypeStruct((np_, din), jnp.float32)),
    )


@functools.lru_cache(maxsize=None)
def _tc_mid(np_, n, din, dh, de):
    def body(p_ref, xs_ref, dinv_ref, w1_ref, b1_ref, g1_ref, be1_ref,
             w2_ref, xs2_ref):
        dinv = dinv_ref[...]
        agg = dinv * (p_ref[0] + p_ref[1] + xs_ref[...])
        z = jnp.dot(agg, w1_ref[...], preferred_element_type=jnp.float32,
                    precision=lax.Precision.HIGHEST)
        z = z + b1_ref[...]
        rows = lax.broadcasted_iota(jnp.int32, (np_, 1), 0)
        r = jnp.where(rows < n, jnp.maximum(z, 0.0), 0.0)
        mu = jnp.sum(r, axis=0, keepdims=True) * (1.0 / n)
        m2 = jnp.sum(r * r, axis=0, keepdims=True) * (1.0 / n)
        var = m2 - mu * mu
        h1 = (r - mu) * (g1_ref[...] * lax.rsqrt(var + _EPS)) + be1_ref[...]
        y2 = jnp.dot(h1.astype(jnp.bfloat16), w2_ref[...].astype(jnp.bfloat16),
                     preferred_element_type=jnp.float32)
        xs2_ref[...] = dinv * y2

    return pl.pallas_call(
        body,
        out_shape=jax.ShapeDtypeStruct((np_, de), jnp.float32),
    )


@functools.lru_cache(maxsize=None)
def _tc_final(np_, n, de, ng):
    win = 512

    def body(p_ref, xs2_ref, dinv_ref, b2_ref, g2_ref, be2_ref, batch_ref,
             wp_ref, bp_ref, out_ref, pooled, h2_buf, lo_smem):
        agg = dinv_ref[...] * (p_ref[0] + p_ref[1] + xs2_ref[...])
        z = agg + b2_ref[...]
        rows = lax.broadcasted_iota(jnp.int32, (np_, 1), 0)
        r = jnp.where(rows < n, jnp.maximum(z, 0.0), 0.0)
        mu = jnp.sum(r, axis=0, keepdims=True) * (1.0 / n)
        m2 = jnp.sum(r * r, axis=0, keepdims=True) * (1.0 / n)
        var = m2 - mu * mu
        h2 = (r - mu) * (g2_ref[...] * lax.rsqrt(var + _EPS)) + be2_ref[...]
        h2_buf[...] = h2
        bids = batch_ref[...]

        gcols = lax.broadcasted_iota(jnp.int32, (1, ng), 1)
        onehot = jnp.where(bids == gcols, 1.0, 0.0)
        cnt = jnp.sum(onehot, axis=0, keepdims=True)
        ri = lax.broadcasted_iota(jnp.int32, (ng, ng), 0)
        ci = lax.broadcasted_iota(jnp.int32, (ng, ng), 1)
        ut = jnp.where(ri < ci, 1.0, 0.0)
        lo = jnp.dot(cnt, ut, preferred_element_type=jnp.float32,
                     precision=lax.Precision.HIGHEST)
        start = jnp.minimum((lo.astype(jnp.int32) // 8) * 8, np_ - win)
        for g in range(ng):
            lo_smem[g] = start[0, g]
        maxcnt = jnp.max(cnt)

        @pl.when(maxcnt <= win - 8.0)
        def _():
            def gbody(g, carry):
                s0 = lo_smem[g]
                wb = batch_ref[pl.ds(s0, win), :]
                wh = h2_buf[pl.ds(s0, win), :]
                m = jnp.where(wb == g, wh, -jnp.inf)
                pooled[pl.ds(g, 1), :] = jnp.max(m, axis=0, keepdims=True)
                return carry

            lax.fori_loop(0, ng, gbody, 0)

        @pl.when(maxcnt > win - 8.0)
        def _():
            def gbody(g, carry):
                m = jnp.where(bids == g, h2_buf[...], -jnp.inf)
                pooled[pl.ds(g, 1), :] = jnp.max(m, axis=0, keepdims=True)
                return carry

            lax.fori_loop(0, ng, gbody, 0)

        out_ref[...] = (
            jnp.dot(pooled[...].astype(jnp.bfloat16),
                    wp_ref[...].astype(jnp.bfloat16),
                    preferred_element_type=jnp.float32)
            + bp_ref[...])

    return pl.pallas_call(
        body,
        out_shape=jax.ShapeDtypeStruct((ng, 1), jnp.float32),
        scratch_shapes=[pltpu.VMEM((ng, de), jnp.float32),
                        pltpu.VMEM((np_, de), jnp.float32),
                        pltpu.SMEM((ng,), jnp.int32)],
    )


def kernel(x, edge_index, batch, W1, b1, g1, be1, W2, b2, g2, be2, Wp, bp):
    n, din = x.shape
    dh = W1.shape[1]
    de = W2.shape[1]
    e = edge_index.shape[1]

    rpt = -(-n // _NS)
    rpt = -(-rpt // 8) * 8
    np_ = rpt * _NS
    cpt = -(-e // (_NW * _CH))
    cpt = cpt + (cpt % 2)
    ep = _NW * cpt * _CH
    pad_rows = np_ - n

    x = x.astype(jnp.float32)
    src = edge_index[0]
    dst = edge_index[1]
    pad_e = ep - e
    spread = n + jnp.arange(pad_e, dtype=jnp.int32) % jnp.int32(pad_rows)
    src_p = jnp.concatenate([src, spread])
    dst_p = jnp.concatenate([dst, spread])
    src3 = src_p.reshape(_NW, cpt, _CH)
    dst3 = dst_p.reshape(_NW, cpt, _CH)

    xp = jnp.zeros((np_, din), jnp.float32).at[:n].set(x)
    batch_p = jnp.concatenate(
        [batch.astype(jnp.int32), jnp.full((pad_rows,), _NG, jnp.int32)]
    ).reshape(np_, 1)

    zeros_d = jnp.zeros((rpt, din), jnp.float32)
    zeros_c = jnp.zeros((rpt, _CH), jnp.float32)
    ones_c = jnp.ones((_CH, _CH), jnp.float32)

    degp = _sc_degree(np_, cpt)(dst3, zeros_c, ones_c)
    dinv, xs = _tc_prep(np_, n, din)(degp, xp)
    p1 = _sc_agg(np_, din, cpt)(xs, src3, dst3, zeros_d)
    xs2 = _tc_mid(np_, n, din, dh, de)(
        p1, xs, dinv, W1, b1.reshape(1, dh), g1.reshape(1, dh),
        be1.reshape(1, dh), W2)
    p2 = _sc_agg(np_, de, cpt)(xs2, src3, dst3, zeros_d)
    out = _tc_final(np_, n, de, _NG)(
        p2, xs2, dinv, b2.reshape(1, de), g2.reshape(1, de),
        be2.reshape(1, de), batch_p, Wp, bp.reshape(1, 1))
    return out
